# baseline (speedup 1.0000x reference)
import functools

import jax
import jax.numpy as jnp
import numpy as np
from jax import lax
from jax.experimental import pallas as pl
from jax.experimental.pallas import tpu as pltpu
from jax.experimental.pallas import tpu_sc as plsc

D_MODEL = 128

_NC = 2
_NS = 16
_NW = _NC * _NS

_G = 64
_C = _G
_NBUF = 4
_PF = 2
_GPF = 2


def _gather_add(xf, idx2, iota, table):
    B = xf.shape[0]
    b_per_w = B // _NW
    n = b_per_w // _C
    assert n % _NBUF == 0

    mesh = plsc.VectorSubcoreMesh(
        core_axis_name="c", subcore_axis_name="s",
        num_cores=_NC, num_subcores=_NS)

    @functools.partial(
        pl.kernel,
        mesh=mesh,
        out_type=jax.ShapeDtypeStruct((B, D_MODEL), jnp.float32),
        scratch_types=[
            pltpu.VMEM((n, _G), jnp.int32),
            pltpu.VMEM((_C,), jnp.int32),
            pltpu.VMEM((_NBUF, _C, D_MODEL), jnp.float32),
            pltpu.VMEM_SHARED((_NS, _NBUF, _C, D_MODEL), jnp.float32),
            [pltpu.SemaphoreType.DMA] * _NBUF,
            [pltpu.SemaphoreType.DMA] * _NBUF,
            [pltpu.SemaphoreType.DMA] * _NBUF,
            [pltpu.SemaphoreType.DMA] * _NBUF,
        ],
    )
    def k(x_hbm, idx_hbm, iota_hbm, table_hbm, out_hbm,
          idx_v, iota_v, gbuf, spm, sem_x, sem_g, sem_a, sem_o):
        sid = lax.axis_index("s")
        wid = sid * _NC + lax.axis_index("c")
        row0 = wid * b_per_w

        pltpu.sync_copy(idx_hbm.at[wid], idx_v)
        pltpu.sync_copy(iota_hbm, iota_v)

        def start_x(c, slot):
            pltpu.async_copy(
                x_hbm.at[pl.ds(row0 + c * _C, _C)], spm.at[sid, slot],
                sem_x[slot])

        def wait_x(c, slot):
            pltpu.make_async_copy(
                x_hbm.at[pl.ds(row0 + c * _C, _C)], spm.at[sid, slot],
                sem_x[slot]).wait()

        def wait_out(c, slot):
            pltpu.make_async_copy(
                spm.at[sid, slot], out_hbm.at[pl.ds(row0 + c * _C, _C)],
                sem_o[slot]).wait()

        def start_gather(c, slot):
            pltpu.async_copy(
                table_hbm.at[idx_v.at[c]], gbuf.at[slot], sem_g[slot])

        def wait_gather(c, slot):
            pltpu.make_async_copy(
                table_hbm.at[idx_v.at[c]], gbuf.at[slot],
                sem_g[slot]).wait()

        def start_add(slot):
            pltpu.async_copy(
                gbuf.at[slot], spm.at[sid, slot].at[iota_v], sem_a[slot],
                add=True)

        def wait_add_start_out(c, slot):
            pltpu.make_async_copy(
                gbuf.at[slot], spm.at[sid, slot].at[iota_v],
                sem_a[slot]).wait()
            pltpu.async_copy(
                spm.at[sid, slot], out_hbm.at[pl.ds(row0 + c * _C, _C)],
                sem_o[slot])

        for b in range(_PF):
            start_x(b, b)
        for b in range(_GPF):
            start_gather(b, b)

        def outer(j, carry):
            for b in range(_NBUF):
                c = j * _NBUF + b

                @pl.when(c < n - _PF)
                def _():
                    slot_n = (b + _PF) % _NBUF

                    def drain_and_fetch():
                        wait_out(c - (_NBUF - _PF), slot_n)
                        start_x(c + _PF, slot_n)

                    if b < _NBUF - _PF:
                        @pl.when(j >= 1)
                        def _():
                            drain_and_fetch()

                        @pl.when(j < 1)
                        def _():
                            start_x(c + _PF, slot_n)
                    else:
                        drain_and_fetch()

                @pl.when(c < n - _GPF)
                def _():
                    start_gather(c + _GPF, (b + _GPF) % _NBUF)

                wait_x(c, b)
                wait_gather(c, b)
                start_add(b)

                if b >= 1:
                    wait_add_start_out(c - 1, b - 1)
                else:
                    @pl.when(j >= 1)
                    def _():
                        wait_add_start_out(c - 1, _NBUF - 1)
            return carry

        lax.fori_loop(0, n // _NBUF, outer, 0)

        wait_add_start_out(n - 1, (n - 1) % _NBUF)
        for i in range(_NBUF):
            c = n - _NBUF + i
            wait_out(c, c % _NBUF)

    return k(xf, idx2, iota, table)


def kernel(x, time, embeddings):
    bt, s, d = x.shape
    b = bt * s
    xf = x.reshape(b, d)
    idx2 = time.reshape(_NW, b // (_NW * _G), _G).astype(jnp.int32)
    iota = jnp.asarray(np.arange(_C, dtype=np.int32))
    out = _gather_add(xf, idx2, iota, embeddings)
    return out.reshape(bt, s, d)


# speedup vs baseline: 1.0126x; 1.0126x over previous
import functools

import jax
import jax.numpy as jnp
import numpy as np
from jax import lax
from jax.experimental import pallas as pl
from jax.experimental.pallas import tpu as pltpu
from jax.experimental.pallas import tpu_sc as plsc

D_MODEL = 128

_NC = 2
_NS = 16
_NW = _NC * _NS

_G = 80
_C = _G
_NBUF = 4
_PF = 2
_GPF = 2


def _gather_add(xf, idx2, iota, table):
    B = xf.shape[0]
    b_per_w = B // _NW
    n = b_per_w // _C
    assert n % _NBUF == 0

    mesh = plsc.VectorSubcoreMesh(
        core_axis_name="c", subcore_axis_name="s",
        num_cores=_NC, num_subcores=_NS)

    @functools.partial(
        pl.kernel,
        mesh=mesh,
        out_type=jax.ShapeDtypeStruct((B, D_MODEL), jnp.float32),
        scratch_types=[
            pltpu.VMEM((n, _G), jnp.int32),
            pltpu.VMEM((_C,), jnp.int32),
            pltpu.VMEM((_NBUF, _C, D_MODEL), jnp.float32),
            pltpu.VMEM_SHARED((_NS, _NBUF, _C, D_MODEL), jnp.float32),
            [pltpu.SemaphoreType.DMA] * _NBUF,
            [pltpu.SemaphoreType.DMA] * _NBUF,
            [pltpu.SemaphoreType.DMA] * _NBUF,
            [pltpu.SemaphoreType.DMA] * _NBUF,
        ],
    )
    def k(x_hbm, idx_hbm, iota_hbm, table_hbm, out_hbm,
          idx_v, iota_v, gbuf, spm, sem_x, sem_g, sem_a, sem_o):
        sid = lax.axis_index("s")
        wid = sid * _NC + lax.axis_index("c")
        row0 = wid * b_per_w

        pltpu.sync_copy(idx_hbm.at[wid], idx_v)
        pltpu.sync_copy(iota_hbm, iota_v)

        def start_x(c, slot):
            pltpu.async_copy(
                x_hbm.at[pl.ds(row0 + c * _C, _C)], spm.at[sid, slot],
                sem_x[slot])

        def wait_x(c, slot):
            pltpu.make_async_copy(
                x_hbm.at[pl.ds(row0 + c * _C, _C)], spm.at[sid, slot],
                sem_x[slot]).wait()

        def wait_out(c, slot):
            pltpu.make_async_copy(
                spm.at[sid, slot], out_hbm.at[pl.ds(row0 + c * _C, _C)],
                sem_o[slot]).wait()

        def start_gather(c, slot):
            pltpu.async_copy(
                table_hbm.at[idx_v.at[c]], gbuf.at[slot], sem_g[slot])

        def wait_gather(c, slot):
            pltpu.make_async_copy(
                table_hbm.at[idx_v.at[c]], gbuf.at[slot],
                sem_g[slot]).wait()

        for b in range(_PF):
            start_x(b, b)
        for b in range(_GPF):
            start_gather(b, b)

        def outer(j, carry):
            for b in range(_NBUF):
                c = j * _NBUF + b

                @pl.when(c < n - _PF)
                def _():
                    slot_n = (b + _PF) % _NBUF

                    def drain_and_fetch():
                        wait_out(c - (_NBUF - _PF), slot_n)
                        start_x(c + _PF, slot_n)

                    if b < _NBUF - _PF:
                        @pl.when(j >= 1)
                        def _():
                            drain_and_fetch()

                        @pl.when(j < 1)
                        def _():
                            start_x(c + _PF, slot_n)
                    else:
                        drain_and_fetch()

                @pl.when(c < n - _GPF)
                def _():
                    start_gather(c + _GPF, (b + _GPF) % _NBUF)

                wait_x(c, b)
                wait_gather(c, b)
                pltpu.async_copy(
                    gbuf.at[b], spm.at[sid, b].at[iota_v], sem_a[b],
                    add=True).wait()
                pltpu.async_copy(
                    spm.at[sid, b], out_hbm.at[pl.ds(row0 + c * _C, _C)],
                    sem_o[b])
            return carry

        lax.fori_loop(0, n // _NBUF, outer, 0)

        for i in range(_NBUF):
            c = n - _NBUF + i
            wait_out(c, c % _NBUF)

    return k(xf, idx2, iota, table)


def kernel(x, time, embeddings):
    bt, s, d = x.shape
    b = bt * s
    xf = x.reshape(b, d)
    idx2 = time.reshape(_NW, b // (_NW * _G), _G).astype(jnp.int32)
    iota = jnp.asarray(np.arange(_C, dtype=np.int32))
    out = _gather_add(xf, idx2, iota, embeddings)
    return out.reshape(bt, s, d)
